# baseline (speedup 1.0000x reference)
import functools

import jax
import jax.numpy as jnp
from jax import lax
from jax.experimental import pallas as pl
from jax.experimental.pallas import tpu as pltpu

ATOM_DIM = 22
HID = 10
HP = 16
NUM_LAYERS = 7
N = 128
G = 128

_DN_STD = (((1,), (0,)), ((), ()))


def _dot(a, b, dn):
    return lax.dot_general(a, b, dn, preferred_element_type=jnp.float32)


def _fused_body(nodes_ref, adj_ref, *refs):
    wc_refs = refs[:NUM_LAYERS]
    b_refs = refs[NUM_LAYERS : 2 * NUM_LAYERS]
    Wf1_ref, bf1_ref, Wf2_ref, bf2_ref, out_ref = refs[2 * NUM_LAYERS :]

    A = adj_ref[...]
    AT = jnp.transpose(A, (0, 2, 1))

    rdeg = jnp.sum(AT, axis=1).reshape(1, G * N)
    rdeg = 1.0 / (rdeg + 1e-6)

    xT = None
    hs = []
    for i in range(NUM_LAYERS):
        Wcat = wc_refs[i][...]
        bT = b_refs[i][...]
        if i == 0:
            tT = _dot(Wcat, nodes_ref[...].reshape(ATOM_DIM, G * N), _DN_STD)
        else:
            tT = _dot(Wcat, xT, _DN_STD)
        yT = tT[:HP]
        zT = tT[HP:]
        aggT = jnp.concatenate(
            [
                _dot(zT[:, g * N : (g + 1) * N], AT[g], _DN_STD)
                for g in range(G)
            ],
            axis=1,
        )
        hT = jnp.maximum(yT + aggT * rdeg + bT, 0.0)
        if i >= (NUM_LAYERS + 1) // 2:
            hT = hT + hs[NUM_LAYERS - 1 - i]
        hs.append(hT)
        xT = hT

    xr = hs[-1].reshape(HP, G, N)
    mx = jnp.max(xr, axis=2)
    mn = jnp.min(xr, axis=2)
    sm = jnp.sum(xr, axis=2)
    av = sm * (1.0 / N)
    featT = jnp.concatenate([mx, mn, av, sm], axis=0)

    h1 = _dot(Wf1_ref[...], featT, _DN_STD) + bf1_ref[...]
    h1 = jnp.where(h1 > 0, h1, jnp.exp(jnp.minimum(h1, 0.0)) - 1.0)
    outG = lax.dot_general(
        h1, Wf2_ref[...], (((0,), (1,)), ((), ())),
        preferred_element_type=jnp.float32,
    )
    out_ref[...] = outG + bf2_ref[...]


@functools.partial(jax.jit, static_argnames=())
def kernel(nodes, adj, W0, b0, W1, b1, W2, b2, W3, b3, W4, b4, W5, b5, W6, b6,
           Wf1, bf1, Wf2, bf2):
    B = nodes.shape[0]
    Ws = [W0, W1, W2, W3, W4, W5, W6]
    bs = [b0, b1, b2, b3, b4, b5, b6]
    dims = [ATOM_DIM] + [HID] * NUM_LAYERS

    w_ops, w_specs = [], []
    for i in range(NUM_LAYERS):
        d = dims[i]
        dp = d if i == 0 else HP
        pad = jnp.zeros((HP - HID, d), jnp.float32)
        wcat = jnp.concatenate([Ws[i][:d].T, pad, Ws[i][d:].T, pad], axis=0)
        wcat = jnp.pad(wcat, ((0, 0), (0, dp - d)))
        w_ops.append(wcat)
        w_specs.append(pl.BlockSpec((2 * HP, dp), lambda i: (0, 0)))
    for i in range(NUM_LAYERS):
        bT = jnp.pad(bs[i], (0, HP - HID)).reshape(HP, 1)
        w_ops.append(bT)
        w_specs.append(pl.BlockSpec((HP, 1), lambda i: (0, 0)))

    wf1e = jnp.pad(
        Wf1.reshape(4, HID, 9), ((0, 0), (0, HP - HID), (0, HP - 9))
    )
    wf1e = wf1e.transpose(2, 0, 1).reshape(HP, 4 * HP)
    bf1e = jnp.pad(bf1, (0, HP - 9)).reshape(HP, 1)
    wf2e = jnp.pad(Wf2.T, ((0, 6), (0, HP - 9)))
    bf2e = jnp.pad(bf2, (0, 6)).reshape(1, 8)
    w_ops += [wf1e, bf1e, wf2e, bf2e]
    w_specs += [
        pl.BlockSpec((HP, 4 * HP), lambda i: (0, 0)),
        pl.BlockSpec((HP, 1), lambda i: (0, 0)),
        pl.BlockSpec((8, HP), lambda i: (0, 0)),
        pl.BlockSpec((1, 8), lambda i: (0, 0)),
    ]

    nodesT = jnp.transpose(nodes, (2, 0, 1))

    grid = (B // G,)
    outG = pl.pallas_call(
        _fused_body,
        grid=grid,
        in_specs=[
            pl.BlockSpec((ATOM_DIM, G, N), lambda i: (0, i, 0)),
            pl.BlockSpec((G, N, N), lambda i: (i, 0, 0)),
            *w_specs,
        ],
        out_specs=pl.BlockSpec((G, 8), lambda i: (i, 0)),
        out_shape=jax.ShapeDtypeStruct((B, 8), jnp.float32),
        compiler_params=pltpu.CompilerParams(
            dimension_semantics=("parallel",),
        ),
    )(nodesT, adj, *w_ops)
    return outG[:, :2]
